# baseline (device time: 15703 ns/iter reference)
import jax
import jax.numpy as jnp
from jax import lax
from jax.experimental import pallas as pl
from jax.experimental.pallas import tpu as pltpu

N_DEV = 8
MASKS = (1, 3, 4)
GRID = 8


def kernel(x):
    m_per, n = x.shape
    assert m_per % GRID == 0
    m_blk = m_per // GRID

    def body(x_ref, out_ref, acc_ref, send_ref, recv_ref, send_sems, recv_sems):
        g = pl.program_id(0)
        my_pos = lax.axis_index("i")

        @pl.when(g == 0)
        def _():
            barrier_sem = pltpu.get_barrier_semaphore()
            for m in MASKS:
                pl.semaphore_signal(
                    barrier_sem,
                    inc=1,
                    device_id=(my_pos ^ m,),
                    device_id_type=pl.DeviceIdType.MESH,
                )
            acc_ref[:, :] = jnp.zeros((1, n), jnp.float32)

        acc_ref[:, :] = acc_ref[:, :] + jnp.sum(
            x_ref[:, :], axis=0, keepdims=True
        )

        @pl.when(g == GRID - 1)
        def _():
            pl.semaphore_wait(pltpu.get_barrier_semaphore(), len(MASKS))
            rdmas = []
            for r, m in enumerate(MASKS):
                send_ref[r, :, :] = acc_ref[:, :]
                rdma = pltpu.make_async_remote_copy(
                    src_ref=send_ref.at[r],
                    dst_ref=recv_ref.at[r],
                    send_sem=send_sems.at[r],
                    recv_sem=recv_sems.at[r],
                    device_id=(my_pos ^ m,),
                    device_id_type=pl.DeviceIdType.MESH,
                )
                rdma.start()
                rdma.wait_recv()
                rdmas.append(rdma)
                acc_ref[:, :] = acc_ref[:, :] + recv_ref[r, :, :]
            out_ref[:, :] = acc_ref[:, :]
            for rdma in rdmas:
                rdma.wait_send()

    return pl.pallas_call(
        body,
        grid=(GRID,),
        out_shape=jax.ShapeDtypeStruct((1, n), jnp.float32),
        in_specs=[
            pl.BlockSpec((m_blk, n), lambda g: (g, 0), memory_space=pltpu.VMEM)
        ],
        out_specs=pl.BlockSpec((1, n), lambda g: (0, 0), memory_space=pltpu.VMEM),
        scratch_shapes=[
            pltpu.VMEM((1, n), jnp.float32),
            pltpu.VMEM((len(MASKS), 1, n), jnp.float32),
            pltpu.VMEM((len(MASKS), 1, n), jnp.float32),
            pltpu.SemaphoreType.DMA((len(MASKS),)),
            pltpu.SemaphoreType.DMA((len(MASKS),)),
        ],
        compiler_params=pltpu.CompilerParams(collective_id=0),
    )(x)


# device time: 13422 ns/iter; 1.1699x vs baseline; 1.1699x over previous
import jax
import jax.numpy as jnp
from jax import lax
from jax.experimental import pallas as pl
from jax.experimental.pallas import tpu as pltpu

N_DEV = 8
MASK_ORDER = (1, 3, 4, 2, 5, 7, 6)
GRID = 8


def kernel(x):
    m_per, n = x.shape
    assert m_per % GRID == 0
    m_blk = m_per // GRID

    def body(x_ref, out_ref, acc_ref, recv_ref, send_sems, recv_sems):
        g = pl.program_id(0)
        my_pos = lax.axis_index("i")

        @pl.when(g == 0)
        def _():
            barrier_sem = pltpu.get_barrier_semaphore()
            for m in MASK_ORDER:
                pl.semaphore_signal(
                    barrier_sem,
                    inc=1,
                    device_id=(my_pos ^ m,),
                    device_id_type=pl.DeviceIdType.MESH,
                )
            acc_ref[:, :] = jnp.zeros((1, n), jnp.float32)

        acc_ref[:, :] = acc_ref[:, :] + jnp.sum(
            x_ref[:, :], axis=0, keepdims=True
        )

        @pl.when(g == GRID - 1)
        def _():
            pl.semaphore_wait(pltpu.get_barrier_semaphore(), len(MASK_ORDER))
            rdmas = []
            for idx, m in enumerate(MASK_ORDER):
                rdma = pltpu.make_async_remote_copy(
                    src_ref=acc_ref,
                    dst_ref=recv_ref.at[idx],
                    send_sem=send_sems.at[idx],
                    recv_sem=recv_sems.at[idx],
                    device_id=(my_pos ^ m,),
                    device_id_type=pl.DeviceIdType.MESH,
                )
                rdma.start()
                rdmas.append(rdma)
            acc = acc_ref[:, :]
            for idx, rdma in enumerate(rdmas):
                rdma.wait_recv()
                acc = acc + recv_ref[idx, :, :]
            out_ref[:, :] = acc
            for rdma in rdmas:
                rdma.wait_send()

    return pl.pallas_call(
        body,
        grid=(GRID,),
        out_shape=jax.ShapeDtypeStruct((1, n), jnp.float32),
        in_specs=[
            pl.BlockSpec((m_blk, n), lambda g: (g, 0), memory_space=pltpu.VMEM)
        ],
        out_specs=pl.BlockSpec((1, n), lambda g: (0, 0), memory_space=pltpu.VMEM),
        scratch_shapes=[
            pltpu.VMEM((1, n), jnp.float32),
            pltpu.VMEM((len(MASK_ORDER), 1, n), jnp.float32),
            pltpu.SemaphoreType.DMA((len(MASK_ORDER),)),
            pltpu.SemaphoreType.DMA((len(MASK_ORDER),)),
        ],
        compiler_params=pltpu.CompilerParams(collective_id=0),
    )(x)
